# baseline (speedup 1.0000x reference)
import functools

import jax
import jax.numpy as jnp
from jax import lax
from jax.experimental import pallas as pl
from jax.experimental.pallas import tpu as pltpu
from jax.experimental.pallas import tpu_sc as plsc

NC, NS = 2, 16
NW = NC * NS
C = 16
LANES = 16


def _make_sc_add(batch, seq_len, d_model):
    seq_per_w = seq_len // NW
    chunks = seq_per_w // C
    vregs = d_model // LANES
    mesh = plsc.VectorSubcoreMesh(core_axis_name="c", subcore_axis_name="s")

    @functools.partial(
        pl.kernel, mesh=mesh,
        out_type=jax.ShapeDtypeStruct((batch * seq_len, d_model), jnp.float32),
        scratch_types=[
            pltpu.VMEM((C, d_model), jnp.float32),
            pltpu.VMEM((C, d_model), jnp.float32),
        ],
    )
    def k(x_hbm, emb_hbm, out_hbm, emb_v, x_v):
        wid = lax.axis_index("s") * NC + lax.axis_index("c")
        s_base = wid * seq_per_w
        for c in range(chunks):
            s0 = s_base + c * C
            pltpu.sync_copy(emb_hbm.at[pl.ds(s0, C)], emb_v)
            for b in range(batch):
                r0 = b * seq_len + s0
                pltpu.sync_copy(x_hbm.at[pl.ds(r0, C)], x_v)

                def row_add(i, _):
                    for j in range(vregs):
                        sl = pl.ds(j * LANES, LANES)
                        x_v[i, sl] = x_v[i, sl] + emb_v[i, sl]
                    return 0

                lax.fori_loop(0, C, row_add, 0)
                pltpu.sync_copy(x_v, out_hbm.at[pl.ds(r0, C)])

    return k


def kernel(x, emb):
    batch, seq_len, d_model = x.shape
    x_flat = x.reshape(batch * seq_len, d_model)
    out = _make_sc_add(batch, seq_len, d_model)(x_flat, emb)
    return out.reshape(batch, seq_len, d_model)


# speedup vs baseline: 1.2916x; 1.2916x over previous
import functools

import jax
import jax.numpy as jnp
from jax import lax
from jax.experimental import pallas as pl
from jax.experimental.pallas import tpu as pltpu
from jax.experimental.pallas import tpu_sc as plsc

NC, NS = 2, 16
NW = NC * NS
C = 8
LANES = 16


def _make_sc_add(batch, seq_len, d_model):
    seq_per_w = seq_len // NW
    chunks = seq_per_w // C
    vregs = d_model // LANES
    mesh = plsc.VectorSubcoreMesh(core_axis_name="c", subcore_axis_name="s")

    @functools.partial(
        pl.kernel, mesh=mesh,
        out_type=jax.ShapeDtypeStruct((batch * seq_len, d_model), jnp.float32),
        scratch_types=[
            pltpu.VMEM((C, d_model), jnp.float32),
        ] + [pltpu.VMEM((C, d_model), jnp.float32) for _ in range(batch)]
          + [pltpu.SemaphoreType.DMA],
    )
    def k(x_hbm, emb_hbm, out_hbm, emb_v, *rest):
        x_vs, sem = rest[:batch], rest[batch]
        wid = lax.axis_index("s") * NC + lax.axis_index("c")
        s_base = wid * seq_per_w
        for c in range(chunks):
            s0 = s_base + c * C
            cp_e = pltpu.async_copy(emb_hbm.at[pl.ds(s0, C)], emb_v, sem)
            cps = [
                pltpu.async_copy(
                    x_hbm.at[pl.ds(b * seq_len + s0, C)], x_vs[b], sem)
                for b in range(batch)
            ]
            cp_e.wait()
            for cp in cps:
                cp.wait()

            def row_add(i, _):
                for j in range(vregs):
                    sl = pl.ds(j * LANES, LANES)
                    e = emb_v[i, sl]
                    for b in range(batch):
                        plsc.addupdate(x_vs[b].at[i, sl], e)
                return 0

            lax.fori_loop(0, C, row_add, 0)
            for b in range(batch):
                pltpu.sync_copy(x_vs[b], out_hbm.at[pl.ds(b * seq_len + s0, C)])

    return k


def kernel(x, emb):
    batch, seq_len, d_model = x.shape
    x_flat = x.reshape(batch * seq_len, d_model)
    out = _make_sc_add(batch, seq_len, d_model)(x_flat, emb)
    return out.reshape(batch, seq_len, d_model)


# speedup vs baseline: 1.8685x; 1.4466x over previous
import functools

import jax
import jax.numpy as jnp
from jax import lax
from jax.experimental import pallas as pl
from jax.experimental.pallas import tpu as pltpu
from jax.experimental.pallas import tpu_sc as plsc

NC, NS = 2, 16
NW = NC * NS
C = 4
NSLOT = 3
LANES = 16


def _make_sc_add(batch, seq_len, d_model):
    seq_per_w = seq_len // NW
    chunks = seq_per_w // C
    assert chunks % NSLOT == 1, "pipeline epilogue assumes chunks % 3 == 1"
    vregs = d_model // LANES
    mesh = plsc.VectorSubcoreMesh(core_axis_name="c", subcore_axis_name="s")

    scratch = (
        [pltpu.VMEM((C, d_model), jnp.float32) for _ in range(NSLOT)]
        + [pltpu.VMEM((C, d_model), jnp.float32)
           for _ in range(NSLOT * batch)]
        + [pltpu.SemaphoreType.DMA for _ in range(2 * NSLOT)]
    )

    @functools.partial(
        pl.kernel, mesh=mesh,
        out_type=jax.ShapeDtypeStruct((batch * seq_len, d_model), jnp.float32),
        scratch_types=scratch,
    )
    def k(x_hbm, emb_hbm, out_hbm, *rest):
        emb_vs = rest[:NSLOT]
        x_vs = [rest[NSLOT + s * batch: NSLOT + (s + 1) * batch]
                for s in range(NSLOT)]
        lsem = rest[NSLOT + NSLOT * batch: NSLOT + NSLOT * batch + NSLOT]
        ssem = rest[NSLOT + NSLOT * batch + NSLOT:]

        wid = lax.axis_index("s") * NC + lax.axis_index("c")
        s_base = wid * seq_per_w

        def issue_loads(slot, c):
            s0 = s_base + c * C
            pltpu.async_copy(emb_hbm.at[pl.ds(s0, C)], emb_vs[slot],
                             lsem[slot])
            for b in range(batch):
                pltpu.async_copy(x_hbm.at[pl.ds(b * seq_len + s0, C)],
                                 x_vs[slot][b], lsem[slot])

        def wait_loads(slot, c):
            s0 = s_base + c * C
            pltpu.make_async_copy(emb_hbm.at[pl.ds(s0, C)], emb_vs[slot],
                                  lsem[slot]).wait()
            for b in range(batch):
                pltpu.make_async_copy(x_hbm.at[pl.ds(b * seq_len + s0, C)],
                                      x_vs[slot][b], lsem[slot]).wait()

        def issue_stores(slot, c):
            s0 = s_base + c * C
            for b in range(batch):
                pltpu.async_copy(x_vs[slot][b],
                                 out_hbm.at[pl.ds(b * seq_len + s0, C)],
                                 ssem[slot])

        def wait_stores(slot, c):
            s0 = s_base + c * C
            for b in range(batch):
                pltpu.make_async_copy(x_vs[slot][b],
                                      out_hbm.at[pl.ds(b * seq_len + s0, C)],
                                      ssem[slot]).wait()

        def compute(slot):
            def row_add(i, _):
                for j in range(vregs):
                    sl = pl.ds(j * LANES, LANES)
                    e = emb_vs[slot][i, sl]
                    for b in range(batch):
                        plsc.addupdate(x_vs[slot][b].at[i, sl], e)
                return 0

            lax.fori_loop(0, C, row_add, 0)

        issue_loads(0, 0)
        issue_loads(1, 1)

        def outer(i, _):
            for kk in range(NSLOT):
                c = i * NSLOT + kk
                wait_loads(kk, c)
                compute(kk)
                issue_stores(kk, c)
                prev = (kk + 2) % NSLOT

                @pl.when(c >= 1)
                def _():
                    wait_stores(prev, c - 1)

                @pl.when(c + 2 < chunks)
                def _():
                    issue_loads(prev, c + 2)
            return 0

        lax.fori_loop(0, (chunks - 1) // NSLOT, outer, 0)

        c_last = chunks - 1
        wait_loads(0, c_last)
        compute(0)
        issue_stores(0, c_last)
        wait_stores(NSLOT - 1, c_last - 1)
        wait_stores(0, c_last)

    return k


def kernel(x, emb):
    batch, seq_len, d_model = x.shape
    x_flat = x.reshape(batch * seq_len, d_model)
    out = _make_sc_add(batch, seq_len, d_model)(x_flat, emb)
    return out.reshape(batch, seq_len, d_model)
